# Initial kernel scaffold; baseline (speedup 1.0000x reference)
import jax
import jax.numpy as jnp
from jax.experimental import pallas as pl


def kernel(x, coords_target, coords_source, W_pe, b_pe, ln_g, ln_b, Wq, Wk, Wv, logit_scale, Wo, bo, W1, b1, W2, b2):
    raise NotImplementedError("write your pallas kernel here")


# speedup vs baseline: 3.4954x; 3.4954x over previous
import functools

import jax
import jax.numpy as jnp
from jax import lax
from jax.experimental import pallas as pl

_NH = 16
_HEADS = 4
_MD = 16
_HD = _MD // _HEADS
_FF = 512
_OUT = 256


def _leaky(u):
    return jnp.where(u >= 0, u, 0.2 * u)


def _tile_kernel(ctT_ref, ct_ref, ct2_ref, cs_ref, cs2_ref, xcsT_ref,
                 wpex_ref, wpey_ref, bpe_ref,
                 lng_ref, lnb_ref, bdq_ref, bdk_ref, wv_ref, scale_ref,
                 bdo_ref, bo_ref, w1_ref, b1_ref, w2_ref, b2_ref, out_ref,
                 *, S, T):
    ctT = ctT_ref[0]
    ct = ct_ref[0]
    ct2 = ct2_ref[0]
    cs = cs_ref[0]
    cs2 = cs2_ref[0]
    xcsT = xcsT_ref[0]

    cross = jnp.dot(ctT, cs, preferred_element_type=jnp.float32)
    d = (ct2 + cs2) - 2.0 * cross

    iota = lax.broadcasted_iota(jnp.int32, (T, S), 1)

    cols = []
    for _ in range(_NH):
        m = jnp.min(d, axis=1, keepdims=True)
        idx = jnp.min(jnp.where(d == m, iota, S), axis=1,
                      keepdims=True)
        sel = iota == idx
        vals = jnp.dot(sel.astype(jnp.float32), xcsT,
                       precision=jax.lax.Precision.HIGHEST,
                       preferred_element_type=jnp.float32)
        cols.append(vals)
        d = jnp.where(sel, jnp.float32(jnp.inf), d)

    g = jnp.concatenate(cols, axis=1)
    gr = g.T.reshape(_NH, 3, T)

    ctx = ct[0:1, :]
    cty = ct[1:2, :]
    x_nh = gr[:, 0, :]
    relx = gr[:, 1, :] - ctx
    rely = gr[:, 2, :] - cty

    wpex = wpex_ref[...][None, :, :]
    wpey = wpey_ref[...][None, :, :]
    bpe = bpe_ref[...][None, :, :]
    pe = jnp.tanh(relx[:, None, :] * wpex + rely[:, None, :] * wpey + bpe)

    h = x_nh[:, None, :] + pe
    mu = jnp.mean(h, axis=1, keepdims=True)
    var = jnp.mean((h - mu) * (h - mu), axis=1, keepdims=True)
    lng = lng_ref[...][None, :, :]
    lnb = lnb_ref[...][None, :, :]
    normed = (h - mu) / jnp.sqrt(var + 1e-5) * lng + lnb
    normed = normed.reshape(_NH * _MD, T)

    q = jnp.dot(bdq_ref[...], normed, preferred_element_type=jnp.float32)
    k = jnp.dot(bdk_ref[...], normed, preferred_element_type=jnp.float32)
    v = (x_nh[:, None, :] * jnp.ones((1, _MD, 1), jnp.float32)
         ).reshape(_NH * _MD, T) * wv_ref[...]

    q_r = q.reshape(_NH, _HEADS, _HD, T)
    k_r = k.reshape(_NH, _HEADS, _HD, T)
    v_r = v.reshape(_NH, _HEADS, _HD, T)

    qn = q_r / (jnp.sqrt(jnp.sum(q_r * q_r, axis=2, keepdims=True)) + 1e-6)
    kn = k_r / (jnp.sqrt(jnp.sum(k_r * k_r, axis=2, keepdims=True)) + 1e-6)

    lg = jnp.zeros((_NH, _NH, _HEADS, T), jnp.float32)
    for j in range(_HD):
        lg = lg + qn[:, None, :, j, :] * kn[None, :, :, j, :]
    lg = lg * scale_ref[...][None, None, :, :]

    mx = jnp.max(lg, axis=1, keepdims=True)
    e = jnp.exp(lg - mx)
    att = e / jnp.sum(e, axis=1, keepdims=True)

    ao = jnp.zeros((_NH, _HEADS, _HD, T), jnp.float32)
    for m_i in range(_NH):
        ao = ao + att[:, m_i, None, :, :].reshape(_NH, _HEADS, 1, T) \
            * v_r[m_i][None, :, :, :]
    ao = ao.reshape(_NH * _MD, T)

    wo_out = jnp.dot(bdo_ref[...], ao,
                     preferred_element_type=jnp.float32) + bo_ref[...]
    x2 = (x_nh[:, None, :] + wo_out.reshape(_NH, _MD, T)) / 2.0
    x2 = x2.reshape(_NH * _MD, T)

    x2t = x2.T
    h1 = _leaky(jnp.dot(x2t, w1_ref[...],
                        preferred_element_type=jnp.float32) + b1_ref[...])
    out = _leaky(jnp.dot(h1, w2_ref[...],
                         preferred_element_type=jnp.float32) + b2_ref[...])
    out_ref[0] = out


def kernel(x, coords_target, coords_source, W_pe, b_pe, ln_g, ln_b, Wq, Wk,
           Wv, logit_scale, Wo, bo, W1, b1, W2, b2):
    b, s, _ = x.shape
    t = coords_target.shape[2]
    T = 128 if t % 128 == 0 else t

    ctT = coords_target.transpose(0, 2, 1)
    ct2_o = jnp.sum(coords_target ** 2, axis=1)[..., None]
    cs2_o = jnp.sum(coords_source ** 2, axis=1)[:, None, :]
    xcsT = jnp.concatenate([x, coords_source.transpose(0, 2, 1)],
                           axis=2)

    eye = jnp.eye(_NH, dtype=jnp.float32)
    bdq = jnp.kron(eye, Wq.T)
    bdk = jnp.kron(eye, Wk.T)
    bdo = jnp.kron(eye, Wo.T)
    wv_tiled = jnp.tile(Wv.reshape(-1), _NH).reshape(_NH * _MD, 1)
    scale = jnp.exp(jnp.minimum(logit_scale, jnp.log(100.0))).reshape(_HEADS, 1)
    bo_full = jnp.tile(bo, _NH).reshape(_NH * _MD, 1)

    wpex = W_pe[0].reshape(_MD, 1)
    wpey = W_pe[1].reshape(_MD, 1)
    bpe = b_pe.reshape(_MD, 1)
    lng = ln_g.reshape(_MD, 1)
    lnb = ln_b.reshape(_MD, 1)
    b1r = b1.reshape(1, _FF)
    b2r = b2.reshape(1, _OUT)

    grid = (b, t // T)
    rep2 = lambda bi, ti: (0, 0)

    out = pl.pallas_call(
        functools.partial(_tile_kernel, S=s, T=T),
        grid=grid,
        in_specs=[
            pl.BlockSpec((1, T, 2), lambda bi, ti: (bi, ti, 0)),
            pl.BlockSpec((1, 2, T), lambda bi, ti: (bi, 0, ti)),
            pl.BlockSpec((1, T, 1), lambda bi, ti: (bi, ti, 0)),
            pl.BlockSpec((1, 2, s), lambda bi, ti: (bi, 0, 0)),
            pl.BlockSpec((1, 1, s), lambda bi, ti: (bi, 0, 0)),
            pl.BlockSpec((1, s, 3), lambda bi, ti: (bi, 0, 0)),
            pl.BlockSpec((_MD, 1), rep2),
            pl.BlockSpec((_MD, 1), rep2),
            pl.BlockSpec((_MD, 1), rep2),
            pl.BlockSpec((_MD, 1), rep2),
            pl.BlockSpec((_MD, 1), rep2),
            pl.BlockSpec((_NH * _MD, _NH * _MD), rep2),
            pl.BlockSpec((_NH * _MD, _NH * _MD), rep2),
            pl.BlockSpec((_NH * _MD, 1), rep2),
            pl.BlockSpec((_HEADS, 1), rep2),
            pl.BlockSpec((_NH * _MD, _NH * _MD), rep2),
            pl.BlockSpec((_NH * _MD, 1), rep2),
            pl.BlockSpec((_NH * _MD, _FF), rep2),
            pl.BlockSpec((1, _FF), rep2),
            pl.BlockSpec((_FF, _OUT), rep2),
            pl.BlockSpec((1, _OUT), rep2),
        ],
        out_specs=pl.BlockSpec((1, T, _OUT), lambda bi, ti: (bi, ti, 0)),
        out_shape=jax.ShapeDtypeStruct((b, t, _OUT), jnp.float32),
    )(ctT, coords_target, ct2_o, coords_source, cs2_o, xcsT,
      wpex, wpey, bpe, lng, lnb, bdq, bdk,
      wv_tiled, scale, bdo, bo_full, W1, b1r, W2, b2r)
    return out


# speedup vs baseline: 10.6035x; 3.0336x over previous
import functools

import jax
import jax.numpy as jnp
from jax import lax
from jax.experimental import pallas as pl

_NH = 16
_HEADS = 4
_MD = 16
_HD = _MD // _HEADS
_FF = 512
_OUT = 256


def _leaky(u):
    return jnp.where(u >= 0, u, 0.2 * u)


def _tile_kernel(ctT_ref, ct_ref, ct2_ref, cs_ref, cs2_ref, xcsT_ref,
                 wpex_ref, wpey_ref, bpe_ref,
                 lng_ref, lnb_ref, bdq_ref, bdk_ref, wv_ref, scale_ref,
                 bdo_ref, bo_ref, w1_ref, b1_ref, w2_ref, b2_ref, out_ref,
                 *, S, T):
    ctT = ctT_ref[0]
    ct = ct_ref[0]
    ct2 = ct2_ref[0]
    cs = cs_ref[0]
    cs2 = cs2_ref[0]
    xcsT = xcsT_ref[0]

    cross = jnp.dot(ctT, cs, preferred_element_type=jnp.float32)
    d = (ct2 + cs2) - 2.0 * cross

    iota = lax.broadcasted_iota(jnp.int32, (T, S), 1)

    cols = []
    for _ in range(_NH):
        m = jnp.min(d, axis=1, keepdims=True)
        idx = jnp.min(jnp.where(d == m, iota, S), axis=1,
                      keepdims=True)
        sel = iota == idx
        vals = jnp.dot(sel.astype(jnp.float32), xcsT,
                       preferred_element_type=jnp.float32)
        cols.append(vals)
        d = jnp.where(sel, jnp.float32(jnp.inf), d)

    g = jnp.concatenate(cols, axis=1)
    gr = g.T.reshape(_NH, 3, T)

    ctx = ct[0:1, :]
    cty = ct[1:2, :]
    x_nh = gr[:, 0, :]
    relx = gr[:, 1, :] - ctx
    rely = gr[:, 2, :] - cty

    wpex = wpex_ref[...][None, :, :]
    wpey = wpey_ref[...][None, :, :]
    bpe = bpe_ref[...][None, :, :]
    pe = jnp.tanh(relx[:, None, :] * wpex + rely[:, None, :] * wpey + bpe)

    h = x_nh[:, None, :] + pe
    mu = jnp.mean(h, axis=1, keepdims=True)
    var = jnp.mean((h - mu) * (h - mu), axis=1, keepdims=True)
    lng = lng_ref[...][None, :, :]
    lnb = lnb_ref[...][None, :, :]
    normed = (h - mu) / jnp.sqrt(var + 1e-5) * lng + lnb
    normed = normed.reshape(_NH * _MD, T)

    q = jnp.dot(bdq_ref[...], normed, preferred_element_type=jnp.float32)
    k = jnp.dot(bdk_ref[...], normed, preferred_element_type=jnp.float32)
    v = (x_nh[:, None, :] * jnp.ones((1, _MD, 1), jnp.float32)
         ).reshape(_NH * _MD, T) * wv_ref[...]

    q_r = q.reshape(_NH, _HEADS, _HD, T)
    k_r = k.reshape(_NH, _HEADS, _HD, T)
    v_r = v.reshape(_NH, _HEADS, _HD, T)

    qn = q_r / (jnp.sqrt(jnp.sum(q_r * q_r, axis=2, keepdims=True)) + 1e-6)
    kn = k_r / (jnp.sqrt(jnp.sum(k_r * k_r, axis=2, keepdims=True)) + 1e-6)

    lg = jnp.zeros((_NH, _NH, _HEADS, T), jnp.float32)
    for j in range(_HD):
        lg = lg + qn[:, None, :, j, :] * kn[None, :, :, j, :]
    lg = lg * scale_ref[...][None, None, :, :]

    mx = jnp.max(lg, axis=1, keepdims=True)
    e = jnp.exp(lg - mx)
    att = e / jnp.sum(e, axis=1, keepdims=True)

    ao = jnp.zeros((_NH, _HEADS, _HD, T), jnp.float32)
    for m_i in range(_NH):
        ao = ao + att[:, m_i, None, :, :].reshape(_NH, _HEADS, 1, T) \
            * v_r[m_i][None, :, :, :]
    ao = ao.reshape(_NH * _MD, T)

    wo_out = jnp.dot(bdo_ref[...], ao,
                     preferred_element_type=jnp.float32) + bo_ref[...]
    x2 = (x_nh[:, None, :] + wo_out.reshape(_NH, _MD, T)) / 2.0
    x2 = x2.reshape(_NH * _MD, T)

    x2t = x2.T
    h1 = _leaky(jnp.dot(x2t, w1_ref[...],
                        preferred_element_type=jnp.float32) + b1_ref[...])
    out = _leaky(jnp.dot(h1, w2_ref[...],
                         preferred_element_type=jnp.float32) + b2_ref[...])
    out_ref[0] = out


def kernel(x, coords_target, coords_source, W_pe, b_pe, ln_g, ln_b, Wq, Wk,
           Wv, logit_scale, Wo, bo, W1, b1, W2, b2):
    b, s, _ = x.shape
    t = coords_target.shape[2]
    T = 128 if t % 128 == 0 else t

    ctT = coords_target.transpose(0, 2, 1)
    ct2_o = jnp.sum(coords_target ** 2, axis=1)[..., None]
    cs2_o = jnp.sum(coords_source ** 2, axis=1)[:, None, :]
    xcsT = jnp.concatenate([x, coords_source.transpose(0, 2, 1)],
                           axis=2)

    eye = jnp.eye(_NH, dtype=jnp.float32)
    bdq = jnp.kron(eye, Wq.T)
    bdk = jnp.kron(eye, Wk.T)
    bdo = jnp.kron(eye, Wo.T)
    wv_tiled = jnp.tile(Wv.reshape(-1), _NH).reshape(_NH * _MD, 1)
    scale = jnp.exp(jnp.minimum(logit_scale, jnp.log(100.0))).reshape(_HEADS, 1)
    bo_full = jnp.tile(bo, _NH).reshape(_NH * _MD, 1)

    wpex = W_pe[0].reshape(_MD, 1)
    wpey = W_pe[1].reshape(_MD, 1)
    bpe = b_pe.reshape(_MD, 1)
    lng = ln_g.reshape(_MD, 1)
    lnb = ln_b.reshape(_MD, 1)
    b1r = b1.reshape(1, _FF)
    b2r = b2.reshape(1, _OUT)

    grid = (b, t // T)
    rep2 = lambda bi, ti: (0, 0)

    out = pl.pallas_call(
        functools.partial(_tile_kernel, S=s, T=T),
        grid=grid,
        in_specs=[
            pl.BlockSpec((1, T, 2), lambda bi, ti: (bi, ti, 0)),
            pl.BlockSpec((1, 2, T), lambda bi, ti: (bi, 0, ti)),
            pl.BlockSpec((1, T, 1), lambda bi, ti: (bi, ti, 0)),
            pl.BlockSpec((1, 2, s), lambda bi, ti: (bi, 0, 0)),
            pl.BlockSpec((1, 1, s), lambda bi, ti: (bi, 0, 0)),
            pl.BlockSpec((1, s, 3), lambda bi, ti: (bi, 0, 0)),
            pl.BlockSpec((_MD, 1), rep2),
            pl.BlockSpec((_MD, 1), rep2),
            pl.BlockSpec((_MD, 1), rep2),
            pl.BlockSpec((_MD, 1), rep2),
            pl.BlockSpec((_MD, 1), rep2),
            pl.BlockSpec((_NH * _MD, _NH * _MD), rep2),
            pl.BlockSpec((_NH * _MD, _NH * _MD), rep2),
            pl.BlockSpec((_NH * _MD, 1), rep2),
            pl.BlockSpec((_HEADS, 1), rep2),
            pl.BlockSpec((_NH * _MD, _NH * _MD), rep2),
            pl.BlockSpec((_NH * _MD, 1), rep2),
            pl.BlockSpec((_NH * _MD, _FF), rep2),
            pl.BlockSpec((1, _FF), rep2),
            pl.BlockSpec((_FF, _OUT), rep2),
            pl.BlockSpec((1, _OUT), rep2),
        ],
        out_specs=pl.BlockSpec((1, T, _OUT), lambda bi, ti: (bi, ti, 0)),
        out_shape=jax.ShapeDtypeStruct((b, t, _OUT), jnp.float32),
    )(ctT, coords_target, ct2_o, coords_source, cs2_o, xcsT,
      wpex, wpey, bpe, lng, lnb, bdq, bdk,
      wv_tiled, scale, bdo, bo_full, W1, b1r, W2, b2r)
    return out
